# Initial kernel scaffold; baseline (speedup 1.0000x reference)
import jax
import jax.numpy as jnp
from jax.experimental import pallas as pl


def kernel(indices, weight):
    raise NotImplementedError("write your pallas kernel here")


# speedup vs baseline: 3.9532x; 3.9532x over previous
import functools

import jax
import jax.numpy as jnp
from jax import lax
from jax.experimental import pallas as pl
from jax.experimental.pallas import tpu as pltpu
from jax.experimental.pallas import tpu_sc as plsc

VOCAB = 100000
D_MODEL = 64
BATCH = 4096
SEQ = 200

N = BATCH * SEQ
NUM_WORKERS = 32
PER_WORKER = N // NUM_WORKERS
CHUNK = 512
NUM_CHUNKS = PER_WORKER // CHUNK

_mesh = plsc.VectorSubcoreMesh(core_axis_name="c", subcore_axis_name="s")


@functools.partial(
    pl.kernel,
    mesh=_mesh,
    out_type=jax.ShapeDtypeStruct((N, D_MODEL), jnp.float32),
    scratch_types=[
        pltpu.VMEM((CHUNK,), jnp.int32),
        pltpu.VMEM((CHUNK, D_MODEL), jnp.float32),
        pltpu.SemaphoreType.DMA,
    ],
    compiler_params=pltpu.CompilerParams(use_tc_tiling_on_sc=False),
)
def _embedding_lookup(idx_hbm, table_hbm, out_hbm, idx_v, rows_v, sem):
    wid = lax.axis_index("s") * 2 + lax.axis_index("c")
    base = wid * PER_WORKER

    def body(g, carry):
        start = base + g * CHUNK
        pltpu.sync_copy(idx_hbm.at[pl.ds(start, CHUNK)], idx_v)
        pltpu.async_copy(table_hbm.at[idx_v], rows_v, sem).wait()
        pltpu.sync_copy(rows_v, out_hbm.at[pl.ds(start, CHUNK)])
        return carry

    lax.fori_loop(0, NUM_CHUNKS, body, 0)


def kernel(indices, weight):
    flat_idx = indices.reshape(N)
    out = _embedding_lookup(flat_idx, weight)
    return out.reshape(BATCH, SEQ, D_MODEL)


# speedup vs baseline: 4.1985x; 1.0621x over previous
import functools

import jax
import jax.numpy as jnp
from jax import lax
from jax.experimental import pallas as pl
from jax.experimental.pallas import tpu as pltpu
from jax.experimental.pallas import tpu_sc as plsc

VOCAB = 100000
D_MODEL = 64
BATCH = 4096
SEQ = 200

N = BATCH * SEQ
NUM_WORKERS = 32
PER_WORKER = N // NUM_WORKERS
CHUNK = 800
NUM_CHUNKS = PER_WORKER // CHUNK

_mesh = plsc.VectorSubcoreMesh(core_axis_name="c", subcore_axis_name="s")


@functools.partial(
    pl.kernel,
    mesh=_mesh,
    out_type=jax.ShapeDtypeStruct((N, D_MODEL), jnp.float32),
    scratch_types=[
        pltpu.VMEM((CHUNK,), jnp.int32),
        pltpu.VMEM((CHUNK,), jnp.int32),
        pltpu.VMEM((CHUNK, D_MODEL), jnp.float32),
        pltpu.VMEM((CHUNK, D_MODEL), jnp.float32),
        pltpu.SemaphoreType.DMA,
        pltpu.SemaphoreType.DMA,
    ],
    compiler_params=pltpu.CompilerParams(use_tc_tiling_on_sc=False),
)
def _embedding_lookup(idx_hbm, table_hbm, out_hbm,
                      idx0, idx1, rows0, rows1, sem0, sem1):
    wid = lax.axis_index("s") * 2 + lax.axis_index("c")
    base = wid * PER_WORKER

    pltpu.sync_copy(idx_hbm.at[pl.ds(base, CHUNK)], idx0)
    pltpu.async_copy(table_hbm.at[idx0], rows0, sem0)

    def body(k, carry):
        g0 = 2 * k
        s1 = base + (g0 + 1) * CHUNK
        pltpu.sync_copy(idx_hbm.at[pl.ds(s1, CHUNK)], idx1)
        pltpu.async_copy(table_hbm.at[idx1], rows1, sem1)
        pltpu.make_async_copy(table_hbm.at[idx0], rows0, sem0).wait()
        pltpu.sync_copy(rows0, out_hbm.at[pl.ds(base + g0 * CHUNK, CHUNK)])
        s2 = base + lax.rem(g0 + 2, NUM_CHUNKS) * CHUNK
        pltpu.sync_copy(idx_hbm.at[pl.ds(s2, CHUNK)], idx0)
        pltpu.async_copy(table_hbm.at[idx0], rows0, sem0)
        pltpu.make_async_copy(table_hbm.at[idx1], rows1, sem1).wait()
        pltpu.sync_copy(rows1, out_hbm.at[pl.ds(s1, CHUNK)])
        return carry

    lax.fori_loop(0, NUM_CHUNKS // 2, body, 0)
    pltpu.make_async_copy(table_hbm.at[idx0], rows0, sem0).wait()


def kernel(indices, weight):
    flat_idx = indices.reshape(N)
    out = _embedding_lookup(flat_idx, weight)
    return out.reshape(BATCH, SEQ, D_MODEL)
